# baseline (device time: 6629 ns/iter reference)
import jax
import jax.numpy as jnp
from jax import lax
from jax.experimental import pallas as pl
from jax.experimental.pallas import tpu as pltpu

NC = 4


def kernel(x):
    m, n = x.shape
    bn = n // NC

    def body(x_ref, out_ref, send_buf, recv_buf, send_sems, recv_sems):
        c = pl.program_id(0)
        my_x = lax.axis_index("x")
        my_y = lax.axis_index("y")
        peer = (1 - my_x, my_y)

        @pl.when(c == 0)
        def _():
            barrier_sem = pltpu.get_barrier_semaphore()
            pl.semaphore_signal(
                barrier_sem, inc=1, device_id=peer,
                device_id_type=pl.DeviceIdType.MESH,
            )
            pl.semaphore_wait(barrier_sem, 1)

        send_buf[c, 0, :] = jnp.max(x_ref[:, :], axis=0)
        rdma = pltpu.make_async_remote_copy(
            src_ref=send_buf.at[c],
            dst_ref=recv_buf.at[c],
            send_sem=send_sems.at[c],
            recv_sem=recv_sems.at[c],
            device_id=peer,
            device_id_type=pl.DeviceIdType.MESH,
        )
        rdma.start()

        @pl.when(c == NC - 1)
        def _():
            for cc in range(NC):
                drain = pltpu.make_async_remote_copy(
                    src_ref=send_buf.at[cc],
                    dst_ref=recv_buf.at[cc],
                    send_sem=send_sems.at[cc],
                    recv_sem=recv_sems.at[cc],
                    device_id=peer,
                    device_id_type=pl.DeviceIdType.MESH,
                )
                drain.wait()
                out_ref[0, pl.ds(cc * bn, bn)] = jnp.maximum(
                    send_buf[cc, 0, :], recv_buf[cc, 0, :]
                )

    return pl.pallas_call(
        body,
        grid=(NC,),
        out_shape=jax.ShapeDtypeStruct((1, n), jnp.float32),
        in_specs=[
            pl.BlockSpec((m, bn), lambda c: (0, c), memory_space=pltpu.VMEM)
        ],
        out_specs=pl.BlockSpec((1, n), lambda c: (0, 0), memory_space=pltpu.VMEM),
        scratch_shapes=[
            pltpu.VMEM((NC, 1, bn), jnp.float32),
            pltpu.VMEM((NC, 1, bn), jnp.float32),
            pltpu.SemaphoreType.DMA((NC,)),
            pltpu.SemaphoreType.DMA((NC,)),
        ],
        compiler_params=pltpu.CompilerParams(collective_id=0),
    )(x)


# device time: 6225 ns/iter; 1.0649x vs baseline; 1.0649x over previous
import jax
import jax.numpy as jnp
from jax import lax
from jax.experimental import pallas as pl
from jax.experimental.pallas import tpu as pltpu

NC = 2


def kernel(x):
    m, n = x.shape
    bn = n // NC

    def body(x_ref, out_ref, send_buf, recv_buf, send_sems, recv_sems):
        c = pl.program_id(0)
        my_x = lax.axis_index("x")
        my_y = lax.axis_index("y")
        peer = (1 - my_x, my_y)

        @pl.when(c == 0)
        def _():
            barrier_sem = pltpu.get_barrier_semaphore()
            pl.semaphore_signal(
                barrier_sem, inc=1, device_id=peer,
                device_id_type=pl.DeviceIdType.MESH,
            )
            pl.semaphore_wait(barrier_sem, 1)

        send_buf[c, 0, :] = jnp.max(x_ref[:, :], axis=0)
        rdma = pltpu.make_async_remote_copy(
            src_ref=send_buf.at[c],
            dst_ref=recv_buf.at[c],
            send_sem=send_sems.at[c],
            recv_sem=recv_sems.at[c],
            device_id=peer,
            device_id_type=pl.DeviceIdType.MESH,
        )
        rdma.start()

        @pl.when(c == NC - 1)
        def _():
            for cc in range(NC):
                drain = pltpu.make_async_remote_copy(
                    src_ref=send_buf.at[cc],
                    dst_ref=recv_buf.at[cc],
                    send_sem=send_sems.at[cc],
                    recv_sem=recv_sems.at[cc],
                    device_id=peer,
                    device_id_type=pl.DeviceIdType.MESH,
                )
                drain.wait()
                out_ref[0, pl.ds(cc * bn, bn)] = jnp.maximum(
                    send_buf[cc, 0, :], recv_buf[cc, 0, :]
                )

    return pl.pallas_call(
        body,
        grid=(NC,),
        out_shape=jax.ShapeDtypeStruct((1, n), jnp.float32),
        in_specs=[
            pl.BlockSpec((m, bn), lambda c: (0, c), memory_space=pltpu.VMEM)
        ],
        out_specs=pl.BlockSpec((1, n), lambda c: (0, 0), memory_space=pltpu.VMEM),
        scratch_shapes=[
            pltpu.VMEM((NC, 1, bn), jnp.float32),
            pltpu.VMEM((NC, 1, bn), jnp.float32),
            pltpu.SemaphoreType.DMA((NC,)),
            pltpu.SemaphoreType.DMA((NC,)),
        ],
        compiler_params=pltpu.CompilerParams(collective_id=0),
    )(x)


# device time: 5881 ns/iter; 1.1272x vs baseline; 1.0585x over previous
import jax
import jax.numpy as jnp
from jax import lax
from jax.experimental import pallas as pl
from jax.experimental.pallas import tpu as pltpu


def kernel(x):
    m, n = x.shape

    def body(x_ref, out_ref, send_buf, recv_buf, send_sem, recv_sem):
        my_x = lax.axis_index("x")
        my_y = lax.axis_index("y")
        peer = (1 - my_x, my_y)

        barrier_sem = pltpu.get_barrier_semaphore()
        pl.semaphore_signal(
            barrier_sem, inc=1, device_id=peer,
            device_id_type=pl.DeviceIdType.MESH,
        )

        send_buf[0, :] = jnp.max(x_ref[:, :], axis=0)

        pl.semaphore_wait(barrier_sem, 1)
        rdma = pltpu.make_async_remote_copy(
            src_ref=send_buf,
            dst_ref=recv_buf,
            send_sem=send_sem,
            recv_sem=recv_sem,
            device_id=peer,
            device_id_type=pl.DeviceIdType.MESH,
        )
        rdma.start()
        rdma.wait()

        out_ref[0, :] = jnp.maximum(send_buf[0, :], recv_buf[0, :])

    return pl.pallas_call(
        body,
        out_shape=jax.ShapeDtypeStruct((1, n), jnp.float32),
        in_specs=[pl.BlockSpec(memory_space=pltpu.VMEM)],
        out_specs=pl.BlockSpec(memory_space=pltpu.VMEM),
        scratch_shapes=[
            pltpu.VMEM((1, n), jnp.float32),
            pltpu.VMEM((1, n), jnp.float32),
            pltpu.SemaphoreType.DMA,
            pltpu.SemaphoreType.DMA,
        ],
        compiler_params=pltpu.CompilerParams(collective_id=0),
    )(x)
